# Initial kernel scaffold; baseline (speedup 1.0000x reference)
import jax
import jax.numpy as jnp
from jax.experimental import pallas as pl


def kernel(x_u, x_v, ci_u, ci_v, edge_u, edge_v, weight):
    raise NotImplementedError("write your pallas kernel here")


# speedup vs baseline: 3.2961x; 3.2961x over previous
import functools

import jax
import jax.numpy as jnp
from jax import lax
from jax.experimental import pallas as pl
from jax.experimental.pallas import tpu as pltpu
from jax.experimental.pallas import tpu_sc as plsc

_U = 10000
_V = 10000
_N = _U + _V
_F = 512
_R = 5
_E = 30000
_ED = _R * _E

_FC = 4
_FW = 128

_NC = 2
_NS = 16
_TPE = _ED // _NS
_B = 125
_BP = 128
_NB = _TPE // _B
_VP = 10240
_DUMMY = _VP
_ACC_ROWS = _VP + 16
_RPT = _VP // _NS


def _mm_body(x_ref, w_ref, o_ref):
    o_ref[...] = jnp.dot(x_ref[...], w_ref[0],
                         preferred_element_type=jnp.float32)


_MBLK = 2000


def _project(x, weight):
    grid = (_N // _MBLK, _R, _FC)
    return pl.pallas_call(
        _mm_body,
        grid=grid,
        in_specs=[
            pl.BlockSpec((_MBLK, _F), lambda m, i, f: (m, 0)),
            pl.BlockSpec((1, _F, _FW), lambda m, i, f: (i, 0, f)),
        ],
        out_specs=pl.BlockSpec(
            (_MBLK, _FW),
            lambda m, i, f: (f * (_R * _N // _MBLK) + i * (_N // _MBLK) + m, 0)),
        out_shape=jax.ShapeDtypeStruct((_FC * _R * _N, _FW), jnp.float32),
    )(x, weight)


def _sc_agg(table, srcs, dsts, zeros):
    mesh = plsc.VectorSubcoreMesh(core_axis_name="c", subcore_axis_name="s")

    @functools.partial(
        pl.kernel,
        mesh=mesh,
        out_type=jax.ShapeDtypeStruct((_NC, _FC, _VP, _FW), jnp.float32),
        scratch_types=[
            pltpu.VMEM((_NB, _BP), jnp.int32),
            pltpu.VMEM((_NB, _BP), jnp.int32),
            pltpu.VMEM((_BP, _FW), jnp.float32),
            pltpu.VMEM_SHARED((_ACC_ROWS, _FW), jnp.float32),
            pltpu.SemaphoreType.DMA,
        ],
    )
    def body(table_h, srcs_h, dsts_h, zeros_h, out_h,
             src_v, dst_v, rows_v, acc, sem):
        c = lax.axis_index("c")
        s = lax.axis_index("s")
        row0 = s * _RPT
        pltpu.sync_copy(dsts_h.at[c, s], dst_v)
        for fc in range(_FC):
            pltpu.sync_copy(zeros_h, acc.at[pl.ds(row0, _RPT)])

            @pl.when(s == _NS - 1)
            def _():
                pltpu.sync_copy(zeros_h.at[pl.ds(0, 16)],
                                acc.at[pl.ds(_VP, 16)])

            pltpu.sync_copy(srcs_h.at[fc, c, s], src_v)
            plsc.subcore_barrier()

            def step(b, _):
                pltpu.async_copy(table_h.at[src_v.at[b]], rows_v, sem).wait()
                pltpu.sync_copy(rows_v, acc.at[dst_v.at[b]], add=True)
                return _

            lax.fori_loop(0, _NB, step, None)
            plsc.subcore_barrier()
            pltpu.sync_copy(acc.at[pl.ds(row0, _RPT)],
                            out_h.at[c, fc, pl.ds(row0, _RPT)])
            plsc.subcore_barrier()

    return body(table, srcs, dsts, zeros)


def _scale_body(a0_ref, a1_ref, cu_ref, cv_ref, hu_ref, hv_ref):
    cv = cv_ref[...]
    cu = cu_ref[...]
    hv_ref[...] = a0_ref[0, 0] * (cv * cv)
    hu_ref[...] = a1_ref[0, 0] * (cu * cu)


def _scale(acc, ci_u, ci_v):
    grid = (_V // _MBLK, _FC)
    return pl.pallas_call(
        _scale_body,
        grid=grid,
        in_specs=[
            pl.BlockSpec((1, 1, _MBLK, _FW), lambda m, f: (0, f, m, 0)),
            pl.BlockSpec((1, 1, _MBLK, _FW), lambda m, f: (1, f, m, 0)),
            pl.BlockSpec((_MBLK, 1), lambda m, f: (m, 0)),
            pl.BlockSpec((_MBLK, 1), lambda m, f: (m, 0)),
        ],
        out_specs=[
            pl.BlockSpec((_MBLK, _FW), lambda m, f: (m, f)),
            pl.BlockSpec((_MBLK, _FW), lambda m, f: (m, f)),
        ],
        out_shape=[
            jax.ShapeDtypeStruct((_U, _F), jnp.float32),
            jax.ShapeDtypeStruct((_V, _F), jnp.float32),
        ],
    )(acc, acc, ci_u, ci_v)


def kernel(x_u, x_v, ci_u, ci_v, edge_u, edge_v, weight):
    x = jnp.concatenate([x_u, x_v], axis=0)
    table = _project(x, weight)

    roff = (jnp.arange(_R, dtype=jnp.int32) * _N)[:, None]
    src_v = (edge_u + roff).reshape(_NS, _NB, _B)
    src_u = (edge_v + roff + _U).reshape(_NS, _NB, _B)
    src = jnp.stack([src_v, src_u])
    src = jnp.pad(src, ((0, 0), (0, 0), (0, 0), (0, _BP - _B)))
    fcoff = (jnp.arange(_FC, dtype=jnp.int32) * (_R * _N)
             ).reshape(_FC, 1, 1, 1, 1)
    srcs = src[None] + fcoff

    dst_v = edge_v.reshape(_NS, _NB, _B)
    dst_u = edge_u.reshape(_NS, _NB, _B)
    dst = jnp.stack([dst_v, dst_u])
    dsts = jnp.pad(dst, ((0, 0), (0, 0), (0, 0), (0, _BP - _B)),
                   constant_values=_DUMMY)

    zeros = jnp.zeros((_RPT, _FW), jnp.float32)
    acc = _sc_agg(table, srcs, dsts, zeros)

    h_u, h_v = _scale(acc, ci_u[:, None], ci_v[:, None])
    return (h_u, h_v)
